# baseline (speedup 1.0000x reference)
import functools

import jax
import jax.numpy as jnp
from jax import lax
from jax.experimental import pallas as pl
from jax.experimental.pallas import tpu as pltpu
from jax.experimental.pallas import tpu_sc as plsc

NL = 16
NC, NS = 2, 16
NW = NC * NS
EB = 128


def _sc_dec(w):
    lo = lax.bitcast_convert_type(lax.shift_left(w, 16), jnp.float32)
    hi = lax.bitcast_convert_type(w & jnp.int32(-65536), jnp.float32)
    return lo, hi


def _sc_enc(a, b):
    ua = lax.bitcast_convert_type(a, jnp.int32)
    ub = lax.bitcast_convert_type(b, jnp.int32)
    ra = lax.shift_right_logical(
        ua + 0x7FFF + (lax.shift_right_logical(ua, 16) & 1), 16)
    rb = (ub + 0x7FFF + (lax.shift_right_logical(ub, 16) & 1)) & jnp.int32(-65536)
    return ra | rb


def _rne16(v):
    odd = lax.shift_right_logical(v, 16) & 1
    return lax.shift_right_logical(v + 0x7FFF + odd, 16)


def _pack_halves(y):
    d2 = y.shape[-1] // 2
    u = lax.bitcast_convert_type(y, jnp.int32)
    lo = _rne16(u[:, :d2])
    hi = _rne16(u[:, d2:])
    return lo | lax.shift_left(hi, 16)


def _unpack_halves(u):
    lo = lax.bitcast_convert_type(lax.shift_left(u, 16), jnp.float32)
    hi = lax.bitcast_convert_type(u & jnp.int32(-65536), jnp.float32)
    return jnp.concatenate([lo, hi], axis=1)


def _node_mm_body(h_ref, wa, ba, wb, bb, wd, bd, we, be, ah, bh, dh, eh):
    x = h_ref[...]
    dn = (((1,), (1,)), ((), ()))
    ah[...] = lax.dot_general(x, wa[...], dn, preferred_element_type=jnp.float32) + ba[...]
    bh[...] = lax.dot_general(x, wb[...], dn, preferred_element_type=jnp.float32) + bb[...]
    dh[...] = lax.dot_general(x, wd[...], dn, preferred_element_type=jnp.float32) + bd[...]
    eh[...] = lax.dot_general(x, we[...], dn, preferred_element_type=jnp.float32) + be[...]


def _node_mm(h, Wa, ba, Wb, bb, Wd, bd, We, be):
    n, d = h.shape
    f32 = jax.ShapeDtypeStruct((n, d), jnp.float32)
    return pl.pallas_call(
        _node_mm_body,
        out_shape=(f32, f32, f32, f32),
    )(h, Wa, ba.reshape(1, d), Wb, bb.reshape(1, d),
      Wd, bd.reshape(1, d), We, be.reshape(1, d))


def _ce_mm_body(e_ref, wc, bc, out):
    dn = (((1,), (1,)), ((), ()))
    out[...] = lax.dot_general(e_ref[...], wc[...], dn,
                               preferred_element_type=jnp.float32) + bc[...]


def _ce_mm(e, Wc, bc, blk=2000):
    E, d = e.shape
    return pl.pallas_call(
        _ce_mm_body,
        grid=(E // blk,),
        in_specs=[
            pl.BlockSpec((blk, d), lambda i: (i, 0)),
            pl.BlockSpec((d, d), lambda i: (0, 0)),
            pl.BlockSpec((1, d), lambda i: (0, 0)),
        ],
        out_specs=pl.BlockSpec((blk, d), lambda i: (i, 0)),
        out_shape=jax.ShapeDtypeStruct((E, d), jnp.float32),
    )(e, Wc, bc.reshape(1, d))


def _sc_pass1(src, dst, Dh, Eh, Ce):
    E, = src.shape
    n, d = Dh.shape
    g = d // NL
    nchunk = E // EB
    kmax = (nchunk + NW - 1) // NW
    tr = 80
    nt = n // tr
    kt = (nt + NS - 1) // NS

    mesh = plsc.VectorSubcoreMesh(core_axis_name="c", subcore_axis_name="s",
                                  num_cores=NC, num_subcores=NS)

    @functools.partial(
        pl.kernel,
        out_type=(
            jax.ShapeDtypeStruct((E, d), jnp.float32),
            jax.ShapeDtypeStruct((NC, n, d), jnp.float32),
        ),
        mesh=mesh,
        scratch_types=[
            pltpu.VMEM((EB,), jnp.int32),
            pltpu.VMEM((EB,), jnp.int32),
            pltpu.VMEM((EB, d), jnp.float32),
            pltpu.VMEM((EB, d), jnp.float32),
            pltpu.VMEM((EB, d), jnp.float32),
            pltpu.VMEM_SHARED((n, d), jnp.float32),
            pltpu.SemaphoreType.DMA,
            pltpu.SemaphoreType.DMA,
        ],
        compiler_params=pltpu.CompilerParams(use_tc_tiling_on_sc=False),
    )
    def k(src_h, dst_h, dh_h, eh_h, ce_h, enew_h, ss_h,
          sidx, didx, gD, gE, ce, shared, sem1, sem2):
        cid = lax.axis_index("c")
        sid = lax.axis_index("s")
        wid = sid * NC + cid

        zv = jnp.zeros((NL,), jnp.float32)

        def zrow(r, _):
            for j in range(d // NL):
                gD[r, pl.ds(j * NL, NL)] = zv
            return 0
        lax.fori_loop(0, tr, zrow, 0)

        for t in range(kt):
            tid = t * NS + sid

            @pl.when(tid < nt)
            def _():
                pltpu.sync_copy(gD.at[pl.ds(0, tr)],
                                shared.at[pl.ds(pl.multiple_of(tid * tr, 8), tr)])
        plsc.subcore_barrier()

        def chunk(kk, _):
            c = kk * NW + wid

            @pl.when(c < nchunk)
            def _():
                base = pl.multiple_of(c * EB, 8)
                pltpu.sync_copy(src_h.at[pl.ds(base, EB)], sidx)
                pltpu.sync_copy(dst_h.at[pl.ds(base, EB)], didx)
                cp1 = pltpu.async_copy(dh_h.at[sidx], gD, sem1)
                cp2 = pltpu.async_copy(eh_h.at[didx], gE, sem2)
                pltpu.sync_copy(ce_h.at[pl.ds(base, EB)], ce)
                cp1.wait()
                cp2.wait()

                @plsc.parallel_loop(0, EB, 1, unroll=4)
                def row(r):
                    for q in range(g):
                        sl = pl.ds(q * NL, NL)
                        x = gD[r, sl] + gE[r, sl] + ce[r, sl]
                        ce[r, sl] = x
                        gD[r, sl] = 1.0 / (1.0 + jnp.exp(-x))

                pltpu.sync_copy(ce, enew_h.at[pl.ds(base, EB)])
                pltpu.sync_copy(gD, shared.at[didx], add=True)
            return 0
        lax.fori_loop(0, kmax, chunk, 0)

        plsc.subcore_barrier()

        for t in range(kt):
            tid = t * NS + sid

            @pl.when(tid < nt)
            def _():
                r0 = pl.multiple_of(tid * tr, 8)
                pltpu.sync_copy(shared.at[pl.ds(r0, tr)], gD.at[pl.ds(0, tr)])
                pltpu.sync_copy(gD.at[pl.ds(0, tr)], ss_h.at[cid, pl.ds(r0, tr)])

    return k(src, dst, Dh, Eh, Ce)


def _sc_pass2(src, dst, enew, eee):
    E, = src.shape
    n, d2 = eee.shape
    d = 2 * d2
    g = d2 // NL
    nchunk = E // EB
    kmax = (nchunk + NW - 1) // NW
    tr = 80
    nt = n // tr
    kt = (nt + NS - 1) // NS

    mesh = plsc.VectorSubcoreMesh(core_axis_name="c", subcore_axis_name="s",
                                  num_cores=NC, num_subcores=NS)

    @functools.partial(
        pl.kernel,
        out_type=jax.ShapeDtypeStruct((NC, n, d), jnp.float32),
        mesh=mesh,
        scratch_types=[
            pltpu.VMEM((EB,), jnp.int32),
            pltpu.VMEM((EB,), jnp.int32),
            pltpu.VMEM((EB, d2), jnp.int32),
            pltpu.VMEM((EB, d), jnp.float32),
            pltpu.VMEM((EB, d), jnp.float32),
            pltpu.VMEM_SHARED((n, d), jnp.float32),
            pltpu.SemaphoreType.DMA,
        ],
        compiler_params=pltpu.CompilerParams(use_tc_tiling_on_sc=False),
    )
    def k(src_h, dst_h, enew_h, eee_h, hacc_h,
          sidx, didx, gB, cb, m, shared, sem1):
        cid = lax.axis_index("c")
        sid = lax.axis_index("s")
        wid = sid * NC + cid

        zv = jnp.zeros((NL,), jnp.float32)

        def zrow(r, _):
            for j in range(d // NL):
                m[r, pl.ds(j * NL, NL)] = zv
            return 0
        lax.fori_loop(0, tr, zrow, 0)

        for t in range(kt):
            tid = t * NS + sid

            @pl.when(tid < nt)
            def _():
                pltpu.sync_copy(m.at[pl.ds(0, tr)],
                                shared.at[pl.ds(pl.multiple_of(tid * tr, 8), tr)])
        plsc.subcore_barrier()

        def chunk(kk, _):
            c = kk * NW + wid

            @pl.when(c < nchunk)
            def _():
                base = pl.multiple_of(c * EB, 8)
                pltpu.sync_copy(src_h.at[pl.ds(base, EB)], sidx)
                pltpu.sync_copy(dst_h.at[pl.ds(base, EB)], didx)
                cp1 = pltpu.async_copy(eee_h.at[sidx], gB, sem1)
                pltpu.sync_copy(enew_h.at[pl.ds(base, EB)], cb)
                cp1.wait()

                @plsc.parallel_loop(0, EB, 1, unroll=4)
                def row(r):
                    for q in range(g):
                        sa = pl.ds(q * NL, NL)
                        sb = pl.ds(d2 + q * NL, NL)
                        ba_, bb_ = _sc_dec(gB[r, sa])
                        m[r, sa] = ba_ / (1.0 + jnp.exp(-cb[r, sa]))
                        m[r, sb] = bb_ / (1.0 + jnp.exp(-cb[r, sb]))

                pltpu.sync_copy(m, shared.at[didx], add=True)
            return 0
        lax.fori_loop(0, kmax, chunk, 0)

        plsc.subcore_barrier()
        for t in range(kt):
            tid = t * NS + sid

            @pl.when(tid < nt)
            def _():
                r0 = pl.multiple_of(tid * tr, 8)
                pltpu.sync_copy(shared.at[pl.ds(r0, tr)], m.at[pl.ds(0, tr)])
                pltpu.sync_copy(m.at[pl.ds(0, tr)], hacc_h.at[cid, pl.ds(r0, tr)])

    return k(src, dst, enew, eee)


def _estats_body(enew_ref, out):
    i = pl.program_id(0)
    x = enew_ref[...]
    s = jnp.sum(x, axis=0, keepdims=True)
    q = jnp.sum(x * x, axis=0, keepdims=True)
    sq = jnp.concatenate([s, q], axis=0)

    @pl.when(i == 0)
    def _():
        out[...] = sq

    @pl.when(i > 0)
    def _():
        out[...] = out[...] + sq


def _tc_estats(enew, blk=2000):
    E, d = enew.shape
    return pl.pallas_call(
        _estats_body,
        grid=(E // blk,),
        in_specs=[pl.BlockSpec((blk, d), lambda i: (i, 0))],
        out_specs=pl.BlockSpec((2, d), lambda i: (0, 0)),
        out_shape=jax.ShapeDtypeStruct((2, d), jnp.float32),
    )(enew)


def _mid_body(ss_ref, est_ref, bh_ref, ge_ref, be_ref, eee, scale, shift, nedges):
    ss = ss_ref[0] + ss_ref[1]
    eee[...] = _pack_halves(bh_ref[...] / (ss + 1e-6))
    st = est_ref[...]
    mean = st[0:1, :] / nedges
    msq = st[1:2, :] / nedges
    var = msq - mean * mean
    sc = ge_ref[...] * lax.rsqrt(var + 1e-5)
    scale[...] = sc
    shift[...] = be_ref[...] - mean * sc


def _tc_mid(ss_part, est, Bh, gamma_e, beta_e, nedges):
    n, d = Bh.shape
    return pl.pallas_call(
        functools.partial(_mid_body, nedges=float(nedges)),
        out_shape=(
            jax.ShapeDtypeStruct((n, d // 2), jnp.int32),
            jax.ShapeDtypeStruct((1, d), jnp.float32),
            jax.ShapeDtypeStruct((1, d), jnp.float32),
        ),
    )(ss_part, est, Bh, gamma_e.reshape(1, d), beta_e.reshape(1, d))


def _eout_body(e_ref, enew_ref, sc_ref, sh_ref, out):
    y = enew_ref[...] * sc_ref[...] + sh_ref[...]
    out[...] = e_ref[...] + jnp.maximum(y, 0.0)


def _tc_eout(e, enew, scale, shift, blk=2000):
    E, d = e.shape
    return pl.pallas_call(
        _eout_body,
        grid=(E // blk,),
        in_specs=[
            pl.BlockSpec((blk, d), lambda i: (i, 0)),
            pl.BlockSpec((blk, d), lambda i: (i, 0)),
            pl.BlockSpec((1, d), lambda i: (0, 0)),
            pl.BlockSpec((1, d), lambda i: (0, 0)),
        ],
        out_specs=pl.BlockSpec((blk, d), lambda i: (i, 0)),
        out_shape=jax.ShapeDtypeStruct((E, d), jnp.float32),
    )(e, enew, scale, shift)


def _hout_body(h_ref, ah_ref, hacc_ref, gh_ref, bh_ref, out):
    hn = ah_ref[...] + hacc_ref[0] + hacc_ref[1]
    mu = jnp.mean(hn, axis=0, keepdims=True)
    var = jnp.mean((hn - mu) ** 2, axis=0, keepdims=True)
    h2 = gh_ref[...] * (hn - mu) * lax.rsqrt(var + 1e-5) + bh_ref[...]
    out[...] = h_ref[...] + jnp.maximum(h2, 0.0)


def _tc_hout(h, Ah, hacc, gamma_h, beta_h):
    n, d = h.shape
    return pl.pallas_call(
        _hout_body,
        out_shape=jax.ShapeDtypeStruct((n, d), jnp.float32),
    )(h, Ah, hacc, gamma_h.reshape(1, d), beta_h.reshape(1, d))


def kernel(h, edge_index, e, Wa, ba, Wb, bb, Wc, bc, Wd, bd, We, be,
           gamma_h, beta_h, gamma_e, beta_e):
    E = e.shape[0]
    src = edge_index[0]
    dst = edge_index[1]

    Ah, Bh, Dh, Eh = _node_mm(h, Wa, ba, Wb, bb, Wd, bd, We, be)
    Ce = _ce_mm(e, Wc, bc)

    e_new, ss_part = _sc_pass1(src, dst, Dh, Eh, Ce)
    est = _tc_estats(e_new)
    eee, scale, shift = _tc_mid(ss_part, est, Bh, gamma_e, beta_e, E)
    hacc = _sc_pass2(src, dst, e_new, eee)

    e_out = _tc_eout(e, e_new, scale, shift)
    h_out = _tc_hout(h, Ah, hacc, gamma_h, beta_h)
    return (h_out, e_out)


# speedup vs baseline: 1.0025x; 1.0025x over previous
import functools

import jax
import jax.numpy as jnp
from jax import lax
from jax.experimental import pallas as pl
from jax.experimental.pallas import tpu as pltpu
from jax.experimental.pallas import tpu_sc as plsc

NL = 16
NC, NS = 2, 16
NW = NC * NS
EB = 128


def _sc_dec(w):
    lo = lax.bitcast_convert_type(lax.shift_left(w, 16), jnp.float32)
    hi = lax.bitcast_convert_type(w & jnp.int32(-65536), jnp.float32)
    return lo, hi


def _rne16(v):
    odd = lax.shift_right_logical(v, 16) & 1
    return lax.shift_right_logical(v + 0x7FFF + odd, 16)


def _pack_halves(y):
    d2 = y.shape[-1] // 2
    u = lax.bitcast_convert_type(y, jnp.int32)
    lo = _rne16(u[:, :d2])
    hi = _rne16(u[:, d2:])
    return lo | lax.shift_left(hi, 16)


def _node_mm_body(h_ref, wa, ba, wb, bb, wd, bd, we, be, ah, bh, dh, eh):
    x = h_ref[...]
    dn = (((1,), (1,)), ((), ()))
    ah[...] = lax.dot_general(x, wa[...], dn, preferred_element_type=jnp.float32) + ba[...]
    bh[...] = lax.dot_general(x, wb[...], dn, preferred_element_type=jnp.float32) + bb[...]
    dh[...] = lax.dot_general(x, wd[...], dn, preferred_element_type=jnp.float32) + bd[...]
    eh[...] = lax.dot_general(x, we[...], dn, preferred_element_type=jnp.float32) + be[...]


def _node_mm(h, Wa, ba, Wb, bb, Wd, bd, We, be):
    n, d = h.shape
    f32 = jax.ShapeDtypeStruct((n, d), jnp.float32)
    return pl.pallas_call(
        _node_mm_body,
        out_shape=(f32, f32, f32, f32),
    )(h, Wa, ba.reshape(1, d), Wb, bb.reshape(1, d),
      Wd, bd.reshape(1, d), We, be.reshape(1, d))


def _ce_mm_body(e_ref, wc, bc, out):
    dn = (((1,), (1,)), ((), ()))
    out[...] = lax.dot_general(e_ref[...], wc[...], dn,
                               preferred_element_type=jnp.float32) + bc[...]


def _ce_mm(e, Wc, bc, blk=2000):
    E, d = e.shape
    return pl.pallas_call(
        _ce_mm_body,
        grid=(E // blk,),
        in_specs=[
            pl.BlockSpec((blk, d), lambda i: (i, 0)),
            pl.BlockSpec((d, d), lambda i: (0, 0)),
            pl.BlockSpec((1, d), lambda i: (0, 0)),
        ],
        out_specs=pl.BlockSpec((blk, d), lambda i: (i, 0)),
        out_shape=jax.ShapeDtypeStruct((E, d), jnp.float32),
    )(e, Wc, bc.reshape(1, d))


def _sc_pass1(src, dst, Dh, Eh, Ce):
    E, = src.shape
    n, d = Dh.shape
    g = d // NL
    nchunk = E // EB
    kmax = (nchunk + NW - 1) // NW
    tr = 80
    nt = n // tr
    kt = (nt + NS - 1) // NS

    mesh = plsc.VectorSubcoreMesh(core_axis_name="c", subcore_axis_name="s",
                                  num_cores=NC, num_subcores=NS)

    @functools.partial(
        pl.kernel,
        out_type=(
            jax.ShapeDtypeStruct((E, d), jnp.float32),
            jax.ShapeDtypeStruct((NC, n, d), jnp.float32),
        ),
        mesh=mesh,
        scratch_types=[
            pltpu.VMEM((EB,), jnp.int32),
            pltpu.VMEM((EB,), jnp.int32),
            pltpu.VMEM((EB, d), jnp.float32),
            pltpu.VMEM((EB, d), jnp.float32),
            pltpu.VMEM((EB, d), jnp.float32),
            pltpu.VMEM_SHARED((n, d), jnp.float32),
            pltpu.SemaphoreType.DMA,
            pltpu.SemaphoreType.DMA,
        ],
        compiler_params=pltpu.CompilerParams(use_tc_tiling_on_sc=False),
    )
    def k(src_h, dst_h, dh_h, eh_h, ce_h, enew_h, ss_h,
          sidx, didx, gD, gE, ce, shared, sem1, sem2):
        cid = lax.axis_index("c")
        sid = lax.axis_index("s")
        wid = sid * NC + cid

        zv = jnp.zeros((NL,), jnp.float32)

        def zrow(r, _):
            for j in range(d // NL):
                gD[r, pl.ds(j * NL, NL)] = zv
            return 0
        lax.fori_loop(0, tr, zrow, 0)

        for t in range(kt):
            tid = t * NS + sid

            @pl.when(tid < nt)
            def _():
                pltpu.sync_copy(gD.at[pl.ds(0, tr)],
                                shared.at[pl.ds(pl.multiple_of(tid * tr, 8), tr)])
        plsc.subcore_barrier()

        def chunk(kk, _):
            c = kk * NW + wid

            @pl.when(c < nchunk)
            def _():
                base = pl.multiple_of(c * EB, 8)
                pltpu.sync_copy(src_h.at[pl.ds(base, EB)], sidx)
                pltpu.sync_copy(dst_h.at[pl.ds(base, EB)], didx)
                cp1 = pltpu.async_copy(dh_h.at[sidx], gD, sem1)
                cp2 = pltpu.async_copy(eh_h.at[didx], gE, sem2)
                pltpu.sync_copy(ce_h.at[pl.ds(base, EB)], ce)
                cp1.wait()
                cp2.wait()

                @plsc.parallel_loop(0, EB, 1, unroll=4)
                def row(r):
                    for q in range(g):
                        sl = pl.ds(q * NL, NL)
                        x = gD[r, sl] + gE[r, sl] + ce[r, sl]
                        ce[r, sl] = x
                        gD[r, sl] = 1.0 / (1.0 + jnp.exp(-x))

                pltpu.sync_copy(ce, enew_h.at[pl.ds(base, EB)])
                pltpu.sync_copy(gD, shared.at[didx], add=True)
            return 0
        lax.fori_loop(0, kmax, chunk, 0)

        plsc.subcore_barrier()

        for t in range(kt):
            tid = t * NS + sid

            @pl.when(tid < nt)
            def _():
                r0 = pl.multiple_of(tid * tr, 8)
                pltpu.sync_copy(shared.at[pl.ds(r0, tr)], gD.at[pl.ds(0, tr)])
                pltpu.sync_copy(gD.at[pl.ds(0, tr)], ss_h.at[cid, pl.ds(r0, tr)])

    return k(src, dst, Dh, Eh, Ce)


def _sc_pass2(src, dst, enew, eee):
    E, = src.shape
    n, d2 = eee.shape
    d = 2 * d2
    g = d2 // NL
    nchunk = E // EB
    kmax = (nchunk + NW - 1) // NW
    tr = 80
    nt = n // tr
    kt = (nt + NS - 1) // NS

    mesh = plsc.VectorSubcoreMesh(core_axis_name="c", subcore_axis_name="s",
                                  num_cores=NC, num_subcores=NS)

    @functools.partial(
        pl.kernel,
        out_type=jax.ShapeDtypeStruct((NC, n, d), jnp.float32),
        mesh=mesh,
        scratch_types=[
            pltpu.VMEM((EB,), jnp.int32),
            pltpu.VMEM((EB,), jnp.int32),
            pltpu.VMEM((EB, d2), jnp.int32),
            pltpu.VMEM((EB, d), jnp.float32),
            pltpu.VMEM((EB, d), jnp.float32),
            pltpu.VMEM_SHARED((n, d), jnp.float32),
            pltpu.SemaphoreType.DMA,
        ],
        compiler_params=pltpu.CompilerParams(use_tc_tiling_on_sc=False),
    )
    def k(src_h, dst_h, enew_h, eee_h, hacc_h,
          sidx, didx, gB, cb, m, shared, sem1):
        cid = lax.axis_index("c")
        sid = lax.axis_index("s")
        wid = sid * NC + cid

        zv = jnp.zeros((NL,), jnp.float32)

        def zrow(r, _):
            for j in range(d // NL):
                m[r, pl.ds(j * NL, NL)] = zv
            return 0
        lax.fori_loop(0, tr, zrow, 0)

        for t in range(kt):
            tid = t * NS + sid

            @pl.when(tid < nt)
            def _():
                pltpu.sync_copy(m.at[pl.ds(0, tr)],
                                shared.at[pl.ds(pl.multiple_of(tid * tr, 8), tr)])
        plsc.subcore_barrier()

        def chunk(kk, _):
            c = kk * NW + wid

            @pl.when(c < nchunk)
            def _():
                base = pl.multiple_of(c * EB, 8)
                pltpu.sync_copy(src_h.at[pl.ds(base, EB)], sidx)
                pltpu.sync_copy(dst_h.at[pl.ds(base, EB)], didx)
                cp1 = pltpu.async_copy(eee_h.at[sidx], gB, sem1)
                pltpu.sync_copy(enew_h.at[pl.ds(base, EB)], cb)
                cp1.wait()

                @plsc.parallel_loop(0, EB, 1, unroll=4)
                def row(r):
                    for q in range(g):
                        sa = pl.ds(q * NL, NL)
                        sb = pl.ds(d2 + q * NL, NL)
                        ba_, bb_ = _sc_dec(gB[r, sa])
                        m[r, sa] = ba_ / (1.0 + jnp.exp(-cb[r, sa]))
                        m[r, sb] = bb_ / (1.0 + jnp.exp(-cb[r, sb]))

                pltpu.sync_copy(m, shared.at[didx], add=True)
            return 0
        lax.fori_loop(0, kmax, chunk, 0)

        plsc.subcore_barrier()
        for t in range(kt):
            tid = t * NS + sid

            @pl.when(tid < nt)
            def _():
                r0 = pl.multiple_of(tid * tr, 8)
                pltpu.sync_copy(shared.at[pl.ds(r0, tr)], m.at[pl.ds(0, tr)])
                pltpu.sync_copy(m.at[pl.ds(0, tr)], hacc_h.at[cid, pl.ds(r0, tr)])

    return k(src, dst, enew, eee)


def _estats_body(enew_ref, out):
    i = pl.program_id(0)
    x = enew_ref[...]
    s = jnp.sum(x, axis=0, keepdims=True)
    q = jnp.sum(x * x, axis=0, keepdims=True)
    sq = jnp.concatenate([s, q], axis=0)

    @pl.when(i == 0)
    def _():
        out[...] = sq

    @pl.when(i > 0)
    def _():
        out[...] = out[...] + sq


def _tc_estats(enew, blk=2000):
    E, d = enew.shape
    return pl.pallas_call(
        _estats_body,
        grid=(E // blk,),
        in_specs=[pl.BlockSpec((blk, d), lambda i: (i, 0))],
        out_specs=pl.BlockSpec((2, d), lambda i: (0, 0)),
        out_shape=jax.ShapeDtypeStruct((2, d), jnp.float32),
    )(enew)


def _mid_body(ss_ref, est_ref, bh_ref, ge_ref, be_ref, eee, scale, shift, nedges):
    ss = ss_ref[0] + ss_ref[1]
    eee[...] = _pack_halves(bh_ref[...] / (ss + 1e-6))
    st = est_ref[...]
    mean = st[0:1, :] / nedges
    msq = st[1:2, :] / nedges
    var = msq - mean * mean
    sc = ge_ref[...] * lax.rsqrt(var + 1e-5)
    scale[...] = sc
    shift[...] = be_ref[...] - mean * sc


def _tc_mid(ss_part, est, Bh, gamma_e, beta_e, nedges):
    n, d = Bh.shape
    return pl.pallas_call(
        functools.partial(_mid_body, nedges=float(nedges)),
        out_shape=(
            jax.ShapeDtypeStruct((n, d // 2), jnp.int32),
            jax.ShapeDtypeStruct((1, d), jnp.float32),
            jax.ShapeDtypeStruct((1, d), jnp.float32),
        ),
    )(ss_part, est, Bh, gamma_e.reshape(1, d), beta_e.reshape(1, d))


def _eout_body(e_ref, enew_ref, sc_ref, sh_ref, out):
    y = enew_ref[...] * sc_ref[...] + sh_ref[...]
    out[...] = e_ref[...] + jnp.maximum(y, 0.0)


def _tc_eout(e, enew, scale, shift, blk=2000):
    E, d = e.shape
    return pl.pallas_call(
        _eout_body,
        grid=(E // blk,),
        in_specs=[
            pl.BlockSpec((blk, d), lambda i: (i, 0)),
            pl.BlockSpec((blk, d), lambda i: (i, 0)),
            pl.BlockSpec((1, d), lambda i: (0, 0)),
            pl.BlockSpec((1, d), lambda i: (0, 0)),
        ],
        out_specs=pl.BlockSpec((blk, d), lambda i: (i, 0)),
        out_shape=jax.ShapeDtypeStruct((E, d), jnp.float32),
    )(e, enew, scale, shift)


def _hout_body(h_ref, ah_ref, hacc_ref, gh_ref, bh_ref, out):
    hn = ah_ref[...] + hacc_ref[0] + hacc_ref[1]
    mu = jnp.mean(hn, axis=0, keepdims=True)
    var = jnp.mean((hn - mu) ** 2, axis=0, keepdims=True)
    h2 = gh_ref[...] * (hn - mu) * lax.rsqrt(var + 1e-5) + bh_ref[...]
    out[...] = h_ref[...] + jnp.maximum(h2, 0.0)


def _tc_hout(h, Ah, hacc, gamma_h, beta_h):
    n, d = h.shape
    return pl.pallas_call(
        _hout_body,
        out_shape=jax.ShapeDtypeStruct((n, d), jnp.float32),
    )(h, Ah, hacc, gamma_h.reshape(1, d), beta_h.reshape(1, d))


def kernel(h, edge_index, e, Wa, ba, Wb, bb, Wc, bc, Wd, bd, We, be,
           gamma_h, beta_h, gamma_e, beta_e):
    E = e.shape[0]
    src = edge_index[0]
    dst = edge_index[1]

    Ah, Bh, Dh, Eh = _node_mm(h, Wa, ba, Wb, bb, Wd, bd, We, be)
    Ce = _ce_mm(e, Wc, bc)

    e_new, ss_part = _sc_pass1(src, dst, Dh, Eh, Ce)
    est = _tc_estats(e_new)
    eee, scale, shift = _tc_mid(ss_part, est, Bh, gamma_e, beta_e, E)
    hacc = _sc_pass2(src, dst, e_new, eee)

    e_out = _tc_eout(e, e_new, scale, shift)
    h_out = _tc_hout(h, Ah, hacc, gamma_h, beta_h)
    return (h_out, e_out)
